# Initial kernel scaffold; baseline (speedup 1.0000x reference)
import jax
import jax.numpy as jnp
from jax.experimental import pallas as pl


def kernel(var, ab):
    raise NotImplementedError("write your pallas kernel here")


# speedup vs baseline: 1.0797x; 1.0797x over previous
import jax
import jax.numpy as jnp
from jax.experimental import pallas as pl
from jax.experimental.pallas import tpu as pltpu


_ROWS_PER_BLK = 512


def _loss_body(var_ref, ab_ref, out_ref):
    j = pl.program_id(1)

    @pl.when(j == 0)
    def _init():
        out_ref[...] = jnp.zeros_like(out_ref)

    v = var_ref[0]
    a = ab_ref[0]
    d = jnp.where(a != 0, v - a, 0.0)
    dd = d * d
    p = jnp.sum(dd, axis=0).reshape(16, 128).sum(axis=0)
    out_ref[0, 0, :] += p


def kernel(var, ab):
    B, R, C = var.shape
    nblk = R // _ROWS_PER_BLK
    partial = pl.pallas_call(
        _loss_body,
        grid=(B, nblk),
        in_specs=[
            pl.BlockSpec((1, _ROWS_PER_BLK, C), lambda b, j: (b, j, 0)),
            pl.BlockSpec((1, _ROWS_PER_BLK, C), lambda b, j: (b, j, 0)),
        ],
        out_specs=pl.BlockSpec((1, 1, 128), lambda b, j: (b, 0, 0)),
        out_shape=jax.ShapeDtypeStruct((B, 1, 128), jnp.float32),
    )(var, ab)
    return jnp.sum(partial, axis=(1, 2))
